# Initial kernel scaffold; baseline (speedup 1.0000x reference)
import jax
import jax.numpy as jnp
from jax.experimental import pallas as pl


def kernel(x, edge_index, W0, a_src0, a_dst0, b0, W1, a_src1, a_dst1, b1):
    raise NotImplementedError("write your pallas kernel here")


# speedup vs baseline: 14.5297x; 14.5297x over previous
import functools

import jax
import jax.numpy as jnp
from jax import lax
from jax.experimental import pallas as pl
from jax.experimental.pallas import tpu as pltpu, tpu_sc as plsc

N = 10000
E = 320000
D = 128
NC = 2
NS = 16
CH = 80
EPT = E // NS
NCHUNK = EPT // CH
ZR = 1000


def _tc1_body(x_ref, w0_ref, w1_ref, as0_ref, ad0_ref, as1_ref, ad1_ref,
              h0_ref, h1_ref, scal_ref):
    x = x_ref[...]
    h0 = jnp.dot(x, w0_ref[...], preferred_element_type=jnp.float32)
    h1 = jnp.dot(x, w1_ref[...], preferred_element_type=jnp.float32)
    h0_ref[...] = h0
    h1_ref[...] = h1
    scal_ref[0, :] = jnp.sum(h0 * as0_ref[...], axis=-1)
    scal_ref[1, :] = jnp.sum(h0 * ad0_ref[...], axis=-1)
    scal_ref[2, :] = jnp.sum(h1 * as1_ref[...], axis=-1)
    scal_ref[3, :] = jnp.sum(h1 * ad1_ref[...], axis=-1)
    scal_ref[4, :] = jnp.zeros_like(scal_ref[4, :])
    scal_ref[5, :] = jnp.zeros_like(scal_ref[5, :])
    scal_ref[6, :] = jnp.zeros_like(scal_ref[6, :])
    scal_ref[7, :] = jnp.zeros_like(scal_ref[7, :])


def _tc1(x, W0, W1, a_src0, a_dst0, a_src1, a_dst1):
    return pl.pallas_call(
        _tc1_body,
        out_shape=[
            jax.ShapeDtypeStruct((N, D), jnp.float32),
            jax.ShapeDtypeStruct((N, D), jnp.float32),
            jax.ShapeDtypeStruct((8, N), jnp.float32),
        ],
    )(x, W0, W1, a_src0, a_dst0, a_src1, a_dst1)


def _sc_body(h0, h1, as0, ad0, as1, ad1, src_h, dst_h,
             out0, out1, dn0, dn1, ss0, ss1, cnt,
             u_sh, dn_sh, ss_sh, cnt_sh,
             srcb, dstb, exf, rows_v, ag, bg, attv, ones_v,
             zbuf, zbufv, sem):
    c = lax.axis_index("c")
    s = lax.axis_index("s")

    def pick(f):
        def run(tbl0, tbl1, *a):
            @pl.when(c == 0)
            def _():
                f(tbl0, *a)

            @pl.when(c == 1)
            def _():
                f(tbl1, *a)
        return run

    zero16 = jnp.zeros((16,), jnp.float32)

    def zfill(r, carry):
        for f in range(D // 16):
            zbuf[r, pl.ds(16 * f, 16)] = zero16
        return carry

    lax.fori_loop(0, 40, zfill, 0)

    def zfillv(i, carry):
        zbufv[pl.ds(i * 16, 16)] = zero16
        return carry

    lax.fori_loop(0, ZR // 16, zfillv, 0)
    zbufv[pl.ds(ZR - 16, 16)] = zero16

    @pl.when(s < 10)
    def _():
        r0 = s * ZR
        for i in range(ZR // 40):
            pltpu.sync_copy(zbuf, u_sh.at[pl.ds(r0 + 40 * i, 40)])
        pltpu.sync_copy(zbufv, dn_sh.at[pl.ds(r0, ZR)])
        pltpu.sync_copy(zbufv, ss_sh.at[pl.ds(r0, ZR)])
        pltpu.sync_copy(zbufv, cnt_sh.at[pl.ds(r0, ZR)])

    for i in range(CH // 16):
        ones_v[pl.ds(16 * i, 16)] = jnp.full((16,), 1.0, jnp.float32)

    plsc.subcore_barrier()

    def g_as(tbl):
        pltpu.async_copy(tbl.at[srcb], ag, sem).wait()

    def g_ad(tbl):
        pltpu.async_copy(tbl.at[dstb], bg, sem).wait()

    def g_rows(tbl):
        pltpu.async_copy(tbl.at[srcb], rows_v, sem).wait()

    gather_as = pick(g_as)
    gather_ad = pick(g_ad)
    gather_rows = pick(g_rows)
    ebase = s * EPT

    def body(j, carry):
        eoff = ebase + j * CH
        pltpu.sync_copy(src_h.at[pl.ds(eoff, CH)], srcb)
        pltpu.sync_copy(dst_h.at[pl.ds(eoff, CH)], dstb)
        gather_as(as0, as1)
        gather_ad(ad0, ad1)
        for i in range(CH // 16):
            sl = pl.ds(16 * i, 16)
            a = ag[sl] + bg[sl]
            a = jnp.maximum(a, 0.2 * a)
            e = jnp.exp(a)
            attv[sl] = e
            exf[pl.ds(j * CH + 16 * i, 16)] = e
        pltpu.sync_copy(attv, dn_sh.at[dstb], add=True)

        @pl.when(c == 0)
        def _():
            pltpu.sync_copy(ones_v, cnt_sh.at[srcb], add=True)

        gather_rows(h0, h1)

        def mbody(g, carry2):
            ev16 = attv[pl.ds(16 * g, 16)]
            for lane in range(16):
                e = ev16[lane]
                r = 16 * g + lane
                for f in range(D // 16):
                    sl = pl.ds(16 * f, 16)
                    rows_v[r, sl] = rows_v[r, sl] * e
            return carry2

        lax.fori_loop(0, CH // 16, mbody, 0)
        pltpu.sync_copy(rows_v, u_sh.at[dstb], add=True)
        return carry

    lax.fori_loop(0, NCHUNK, body, 0)
    plsc.subcore_barrier()

    def body2(j, carry):
        eoff = ebase + j * CH
        pltpu.sync_copy(src_h.at[pl.ds(eoff, CH)], srcb)
        pltpu.sync_copy(dst_h.at[pl.ds(eoff, CH)], dstb)
        pltpu.async_copy(dn_sh.at[dstb], bg, sem).wait()
        for i in range(CH // 16):
            sl = pl.ds(16 * i, 16)
            attv[sl] = exf[pl.ds(j * CH + 16 * i, 16)] / (bg[sl] + 1e-16)
        pltpu.sync_copy(attv, ss_sh.at[srcb], add=True)
        return carry

    lax.fori_loop(0, NCHUNK, body2, 0)
    plsc.subcore_barrier()

    @pl.when(s < 10)
    def _():
        r0 = s * ZR

        def d_out(o):
            def dchunk(i, carry):
                sl = pl.ds(r0 + 40 * i, 40)
                pltpu.sync_copy(u_sh.at[sl], zbuf)
                pltpu.sync_copy(zbuf, o.at[sl])
                return carry

            lax.fori_loop(0, ZR // 40, dchunk, 0)

        def d_vec(sh, o):
            sl = pl.ds(r0, ZR)
            pltpu.sync_copy(sh.at[sl], zbufv)
            pltpu.sync_copy(zbufv, o.at[sl])

        pick(d_out)(out0, out1)
        pick(lambda o: d_vec(dn_sh, o))(dn0, dn1)
        pick(lambda o: d_vec(ss_sh, o))(ss0, ss1)

        @pl.when(c == 0)
        def _():
            d_vec(cnt_sh, cnt)


def _sc_call(h0, h1, as0, ad0, as1, ad1, src_r, dst_r):
    mesh = plsc.VectorSubcoreMesh(core_axis_name="c", subcore_axis_name="s",
                                  num_cores=NC, num_subcores=NS)
    f32 = jnp.float32
    out_type = [
        jax.ShapeDtypeStruct((N, D), f32),
        jax.ShapeDtypeStruct((N, D), f32),
        jax.ShapeDtypeStruct((N,), f32),
        jax.ShapeDtypeStruct((N,), f32),
        jax.ShapeDtypeStruct((N,), f32),
        jax.ShapeDtypeStruct((N,), f32),
        jax.ShapeDtypeStruct((N,), f32),
    ]
    scratch = [
        pltpu.VMEM_SHARED((N, D), f32),
        pltpu.VMEM_SHARED((N,), f32),
        pltpu.VMEM_SHARED((N,), f32),
        pltpu.VMEM_SHARED((N,), f32),
        pltpu.VMEM((CH,), jnp.int32),
        pltpu.VMEM((CH,), jnp.int32),
        pltpu.VMEM((EPT,), f32),
        pltpu.VMEM((CH, D), f32),
        pltpu.VMEM((CH,), f32),
        pltpu.VMEM((CH,), f32),
        pltpu.VMEM((CH,), f32),
        pltpu.VMEM((CH,), f32),
        pltpu.VMEM((40, D), f32),
        pltpu.VMEM((ZR,), f32),
        pltpu.SemaphoreType.DMA,
    ]
    fn = pl.kernel(_sc_body, out_type=out_type, mesh=mesh,
                   scratch_types=scratch)
    return fn(h0, h1, as0, ad0, as1, ad1, src_r, dst_r)


def _tc2_body(o0_ref, o1_ref, dn0_ref, dn1_ref, ss0_ref, ss1_ref, cnt_ref,
              b0_ref, b1_ref, feat_ref, scores_ref):
    eps = 1e-16
    o0 = o0_ref[...] / (dn0_ref[...] + eps) + b0_ref[...]
    o1 = o1_ref[...] / (dn1_ref[...] + eps) + b1_ref[...]
    f0 = jnp.maximum(o0, 0.01 * o0)
    f1 = jnp.maximum(o1, 0.01 * o1)
    feat_ref[...] = f0 + f1
    ssum = ss0_ref[...] + ss1_ref[...]
    scores_ref[...] = ssum / jnp.maximum(cnt_ref[...], 1.0)


def _tc2(out0, out1, dn0, dn1, ss0, ss1, cnt, b0, b1):
    f32 = jnp.float32
    return pl.pallas_call(
        _tc2_body,
        out_shape=[
            jax.ShapeDtypeStruct((N, D), f32),
            jax.ShapeDtypeStruct((N, 1), f32),
        ],
    )(out0, out1, dn0, dn1, ss0, ss1, cnt, b0, b1)


def kernel(x, edge_index, W0, a_src0, a_dst0, b0, W1, a_src1, a_dst1, b1):
    src_r = edge_index[0]
    dst_r = edge_index[1]
    h0, h1, scal = _tc1(x, W0, W1,
                        a_src0.reshape(1, D), a_dst0.reshape(1, D),
                        a_src1.reshape(1, D), a_dst1.reshape(1, D))
    as0, ad0, as1, ad1 = scal[0], scal[1], scal[2], scal[3]
    out0, out1, dn0, dn1, ss0, ss1, cnt = _sc_call(
        h0, h1, as0, ad0, as1, ad1, src_r, dst_r)
    feat, scores = _tc2(out0, out1,
                        dn0.reshape(N, 1), dn1.reshape(N, 1),
                        ss0.reshape(N, 1), ss1.reshape(N, 1),
                        cnt.reshape(N, 1),
                        b0.reshape(1, D), b1.reshape(1, D))
    return (feat, scores.reshape(N))


# speedup vs baseline: 33.1402x; 2.2809x over previous
import jax
import jax.numpy as jnp
from jax import lax
from jax.experimental import pallas as pl
from jax.experimental.pallas import tpu as pltpu, tpu_sc as plsc

N = 10000
E = 320000
D = 128
NC = 2
NS = 16
CH = 80
BLK = 50
EPT = E // NS
NCHUNK = EPT // CH
NBLK = NCHUNK // BLK
ZR = 1000


def _tc1_body(x_ref, w0_ref, w1_ref, as0_ref, ad0_ref, as1_ref, ad1_ref,
              h0_ref, h1_ref, scal_ref):
    x = x_ref[...]
    h0 = jnp.dot(x, w0_ref[...], preferred_element_type=jnp.float32)
    h1 = jnp.dot(x, w1_ref[...], preferred_element_type=jnp.float32)
    h0_ref[...] = h0
    h1_ref[...] = h1
    scal_ref[0, :] = jnp.sum(h0 * as0_ref[...], axis=-1)
    scal_ref[1, :] = jnp.sum(h0 * ad0_ref[...], axis=-1)
    scal_ref[2, :] = jnp.sum(h1 * as1_ref[...], axis=-1)
    scal_ref[3, :] = jnp.sum(h1 * ad1_ref[...], axis=-1)
    scal_ref[4, :] = jnp.zeros_like(scal_ref[4, :])
    scal_ref[5, :] = jnp.zeros_like(scal_ref[5, :])
    scal_ref[6, :] = jnp.zeros_like(scal_ref[6, :])
    scal_ref[7, :] = jnp.zeros_like(scal_ref[7, :])


def _tc1(x, W0, W1, a_src0, a_dst0, a_src1, a_dst1):
    return pl.pallas_call(
        _tc1_body,
        out_shape=[
            jax.ShapeDtypeStruct((N, D), jnp.float32),
            jax.ShapeDtypeStruct((N, D), jnp.float32),
            jax.ShapeDtypeStruct((8, N), jnp.float32),
        ],
    )(x, W0, W1, a_src0, a_dst0, a_src1, a_dst1)


def _sc_body(h0, h1, as0, ad0, as1, ad1, src_h, dst_h,
             out0, out1, dn0, dn1, ss0, ss1, cnt, exh,
             u_sh, dn_sh, ss_sh, cnt_sh,
             srcb, dstb, rows_a, rows_b, aga, bga, agb, bgb,
             atta, attb, exb, ones_v, zbuf, zbufv,
             sg0, sg1, sem):
    c = lax.axis_index("c")
    s = lax.axis_index("s")

    def pick(f):
        def run(tbl0, tbl1, *a):
            @pl.when(c == 0)
            def _():
                f(tbl0, *a)

            @pl.when(c == 1)
            def _():
                f(tbl1, *a)
        return run

    zero16 = jnp.zeros((16,), jnp.float32)

    def zfill(r, carry):
        for f in range(D // 16):
            zbuf[r, pl.ds(16 * f, 16)] = zero16
        return carry

    lax.fori_loop(0, 40, zfill, 0)

    def zfillv(i, carry):
        zbufv[pl.ds(i * 16, 16)] = zero16
        return carry

    lax.fori_loop(0, ZR // 16, zfillv, 0)
    zbufv[pl.ds(ZR - 16, 16)] = zero16

    @pl.when(s < 10)
    def _():
        r0 = s * ZR
        for i in range(ZR // 40):
            pltpu.sync_copy(zbuf, u_sh.at[pl.ds(r0 + 40 * i, 40)])
        pltpu.sync_copy(zbufv, dn_sh.at[pl.ds(r0, ZR)])
        pltpu.sync_copy(zbufv, ss_sh.at[pl.ds(r0, ZR)])
        pltpu.sync_copy(zbufv, cnt_sh.at[pl.ds(r0, ZR)])

    for i in range(CH // 16):
        ones_v[pl.ds(16 * i, 16)] = jnp.full((16,), 1.0, jnp.float32)

    plsc.subcore_barrier()

    ebase = s * EPT
    exbase = c * E + ebase

    def start_g(x, ag, bg, rows, sg):
        def go(tas, tad, th):
            pltpu.async_copy(tas.at[srcb.at[x]], ag, sg)
            pltpu.async_copy(tad.at[dstb.at[x]], bg, sg)
            pltpu.async_copy(th.at[srcb.at[x]], rows, sg)

        @pl.when(c == 0)
        def _():
            go(as0, ad0, h0)

        @pl.when(c == 1)
        def _():
            go(as1, ad1, h1)

    def wait_g(x, ag, bg, rows, sg):
        def wg(tas, tad, th):
            pltpu.make_async_copy(tas.at[srcb.at[x]], ag, sg).wait()
            pltpu.make_async_copy(tad.at[dstb.at[x]], bg, sg).wait()
            pltpu.make_async_copy(th.at[srcb.at[x]], rows, sg).wait()

        @pl.when(c == 0)
        def _():
            wg(as0, ad0, h0)

        @pl.when(c == 1)
        def _():
            wg(as1, ad1, h1)

    def compute_ex(ag, bg, att):
        for i in range(CH // 16):
            sl = pl.ds(16 * i, 16)
            a = ag[sl] + bg[sl]
            a = jnp.maximum(a, 0.2 * a)
            att[sl] = jnp.exp(a)

    def mul_rows(rows, att):
        def mbody(g, carry2):
            ev16 = att[pl.ds(16 * g, 16)]
            for lane in range(16):
                e = ev16[lane]
                r = 16 * g + lane
                for f in range(D // 16):
                    sl = pl.ds(16 * f, 16)
                    rows[r, sl] = rows[r, sl] * e
            return carry2

        lax.fori_loop(0, CH // 16, mbody, 0)

    def scatter_sc(bgl, x, rows, att):
        pltpu.sync_copy(rows, u_sh.at[dstb.at[x]], add=True)
        pltpu.sync_copy(att, dn_sh.at[dstb.at[x]], add=True)
        pltpu.sync_copy(att, exh.at[pl.ds(exbase + bgl * CH, CH)])

        @pl.when(c == 0)
        def _():
            pltpu.sync_copy(ones_v, cnt_sh.at[srcb.at[x]], add=True)

    def block(b, carry):
        pltpu.sync_copy(src_h.at[s * NBLK + b], srcb)
        pltpu.sync_copy(dst_h.at[s * NBLK + b], dstb)
        base = b * BLK
        start_g(0, aga, bga, rows_a, sg0)

        def pair(k, carry2):
            a = 2 * k
            bl = 2 * k + 1
            wait_g(a, aga, bga, rows_a, sg0)
            compute_ex(aga, bga, atta)
            start_g(bl, agb, bgb, rows_b, sg1)
            mul_rows(rows_a, atta)
            scatter_sc(base + a, a, rows_a, atta)
            wait_g(bl, agb, bgb, rows_b, sg1)
            compute_ex(agb, bgb, attb)

            @pl.when(k < BLK // 2 - 1)
            def _():
                start_g(a + 2, aga, bga, rows_a, sg0)

            mul_rows(rows_b, attb)
            scatter_sc(base + bl, bl, rows_b, attb)
            return carry2

        lax.fori_loop(0, BLK // 2, pair, 0)
        return carry

    lax.fori_loop(0, NBLK, block, 0)
    plsc.subcore_barrier()

    def block2(b, carry):
        pltpu.sync_copy(src_h.at[s * NBLK + b], srcb)
        pltpu.sync_copy(dst_h.at[s * NBLK + b], dstb)
        base = b * BLK

        def body2(k, carry2):
            off = exbase + (base + k) * CH
            pltpu.sync_copy(exh.at[pl.ds(off, CH)], exb)
            pltpu.async_copy(dn_sh.at[dstb.at[k]], bga, sem).wait()
            for i in range(CH // 16):
                sl = pl.ds(16 * i, 16)
                atta[sl] = exb[sl] / (bga[sl] + 1e-16)
            pltpu.sync_copy(atta, ss_sh.at[srcb.at[k]], add=True)
            return carry2

        lax.fori_loop(0, BLK, body2, 0)
        return carry

    lax.fori_loop(0, NBLK, block2, 0)
    plsc.subcore_barrier()

    @pl.when(s < 10)
    def _():
        r0 = s * ZR

        def d_out(o):
            def dchunk(i, carry):
                sl = pl.ds(r0 + 40 * i, 40)
                pltpu.sync_copy(u_sh.at[sl], zbuf)
                pltpu.sync_copy(zbuf, o.at[sl])
                return carry

            lax.fori_loop(0, ZR // 40, dchunk, 0)

        def d_vec(sh, o):
            sl = pl.ds(r0, ZR)
            pltpu.sync_copy(sh.at[sl], zbufv)
            pltpu.sync_copy(zbufv, o.at[sl])

        pick(d_out)(out0, out1)
        pick(lambda o: d_vec(dn_sh, o))(dn0, dn1)
        pick(lambda o: d_vec(ss_sh, o))(ss0, ss1)

        @pl.when(c == 0)
        def _():
            d_vec(cnt_sh, cnt)


def _sc_call(h0, h1, as0, ad0, as1, ad1, src_r, dst_r):
    mesh = plsc.VectorSubcoreMesh(core_axis_name="c", subcore_axis_name="s",
                                  num_cores=NC, num_subcores=NS)
    f32 = jnp.float32
    out_type = [
        jax.ShapeDtypeStruct((N, D), f32),
        jax.ShapeDtypeStruct((N, D), f32),
        jax.ShapeDtypeStruct((N,), f32),
        jax.ShapeDtypeStruct((N,), f32),
        jax.ShapeDtypeStruct((N,), f32),
        jax.ShapeDtypeStruct((N,), f32),
        jax.ShapeDtypeStruct((N,), f32),
        jax.ShapeDtypeStruct((2 * E,), f32),
    ]
    scratch = [
        pltpu.VMEM_SHARED((N, D), f32),
        pltpu.VMEM_SHARED((N,), f32),
        pltpu.VMEM_SHARED((N,), f32),
        pltpu.VMEM_SHARED((N,), f32),
        pltpu.VMEM((BLK, CH), jnp.int32),
        pltpu.VMEM((BLK, CH), jnp.int32),
        pltpu.VMEM((CH, D), f32),
        pltpu.VMEM((CH, D), f32),
        pltpu.VMEM((CH,), f32),
        pltpu.VMEM((CH,), f32),
        pltpu.VMEM((CH,), f32),
        pltpu.VMEM((CH,), f32),
        pltpu.VMEM((CH,), f32),
        pltpu.VMEM((CH,), f32),
        pltpu.VMEM((CH,), f32),
        pltpu.VMEM((CH,), f32),
        pltpu.VMEM((40, D), f32),
        pltpu.VMEM((ZR,), f32),
        pltpu.SemaphoreType.DMA,
        pltpu.SemaphoreType.DMA,
        pltpu.SemaphoreType.DMA,
    ]
    fn = pl.kernel(_sc_body, out_type=out_type, mesh=mesh,
                   scratch_types=scratch)
    return fn(h0, h1, as0, ad0, as1, ad1, src_r, dst_r)


def _tc2_body(o0_ref, o1_ref, dn0_ref, dn1_ref, ss0_ref, ss1_ref, cnt_ref,
              b0_ref, b1_ref, feat_ref, scores_ref):
    eps = 1e-16
    o0 = o0_ref[...] / (dn0_ref[...] + eps) + b0_ref[...]
    o1 = o1_ref[...] / (dn1_ref[...] + eps) + b1_ref[...]
    f0 = jnp.maximum(o0, 0.01 * o0)
    f1 = jnp.maximum(o1, 0.01 * o1)
    feat_ref[...] = f0 + f1
    ssum = ss0_ref[...] + ss1_ref[...]
    scores_ref[...] = ssum / jnp.maximum(cnt_ref[...], 1.0)


def _tc2(out0, out1, dn0, dn1, ss0, ss1, cnt, b0, b1):
    f32 = jnp.float32
    return pl.pallas_call(
        _tc2_body,
        out_shape=[
            jax.ShapeDtypeStruct((N, D), f32),
            jax.ShapeDtypeStruct((N, 1), f32),
        ],
    )(out0, out1, dn0, dn1, ss0, ss1, cnt, b0, b1)


def kernel(x, edge_index, W0, a_src0, a_dst0, b0, W1, a_src1, a_dst1, b1):
    src_r = edge_index[0].reshape(NS * NBLK, BLK, CH)
    dst_r = edge_index[1].reshape(NS * NBLK, BLK, CH)
    h0, h1, scal = _tc1(x, W0, W1,
                        a_src0.reshape(1, D), a_dst0.reshape(1, D),
                        a_src1.reshape(1, D), a_dst1.reshape(1, D))
    as0, ad0, as1, ad1 = scal[0], scal[1], scal[2], scal[3]
    out0, out1, dn0, dn1, ss0, ss1, cnt, _ = _sc_call(
        h0, h1, as0, ad0, as1, ad1, src_r, dst_r)
    feat, scores = _tc2(out0, out1,
                        dn0.reshape(N, 1), dn1.reshape(N, 1),
                        ss0.reshape(N, 1), ss1.reshape(N, 1),
                        cnt.reshape(N, 1),
                        b0.reshape(1, D), b1.reshape(1, D))
    return (feat, scores.reshape(N))


# speedup vs baseline: 37.7375x; 1.1387x over previous
import jax
import jax.numpy as jnp
from jax import lax
from jax.experimental import pallas as pl
from jax.experimental.pallas import tpu as pltpu, tpu_sc as plsc

N = 10000
E = 320000
D = 128
NC = 2
NS = 16
CH = 80
BLK = 50
EPT = E // NS
NCHUNK = EPT // CH
NBLK = NCHUNK // BLK
ZR = 1000


def _tc1_body(x_ref, w0_ref, w1_ref, as0_ref, ad0_ref, as1_ref, ad1_ref,
              h0_ref, h1_ref, scal_ref):
    x = x_ref[...]
    h0 = jnp.dot(x, w0_ref[...], preferred_element_type=jnp.float32)
    h1 = jnp.dot(x, w1_ref[...], preferred_element_type=jnp.float32)
    h0_ref[...] = h0
    h1_ref[...] = h1
    scal_ref[0, :] = jnp.sum(h0 * as0_ref[...], axis=-1)
    scal_ref[1, :] = jnp.sum(h0 * ad0_ref[...], axis=-1)
    scal_ref[2, :] = jnp.sum(h1 * as1_ref[...], axis=-1)
    scal_ref[3, :] = jnp.sum(h1 * ad1_ref[...], axis=-1)
    scal_ref[4, :] = jnp.zeros_like(scal_ref[4, :])
    scal_ref[5, :] = jnp.zeros_like(scal_ref[5, :])
    scal_ref[6, :] = jnp.zeros_like(scal_ref[6, :])
    scal_ref[7, :] = jnp.zeros_like(scal_ref[7, :])


def _tc1(x, W0, W1, a_src0, a_dst0, a_src1, a_dst1):
    return pl.pallas_call(
        _tc1_body,
        out_shape=[
            jax.ShapeDtypeStruct((N, D), jnp.float32),
            jax.ShapeDtypeStruct((N, D), jnp.float32),
            jax.ShapeDtypeStruct((8, N), jnp.float32),
        ],
    )(x, W0, W1, a_src0, a_dst0, a_src1, a_dst1)


def _sc_body(h0, h1, as0, ad0, as1, ad1, src_h, dst_h,
             out0, out1, dn0, dn1, ss0, ss1, cnt, exh,
             u_sh, dn_sh, ss_sh, cnt_sh,
             srcb, dstb, rows_a, rows_b, aga, bga, agb, bgb,
             atta, attb, exb, ones_v, zbuf, zbufv,
             sg0, sg1, sem):
    c = lax.axis_index("c")
    s = lax.axis_index("s")

    def pick(f):
        def run(tbl0, tbl1, *a):
            @pl.when(c == 0)
            def _():
                f(tbl0, *a)

            @pl.when(c == 1)
            def _():
                f(tbl1, *a)
        return run

    zero16 = jnp.zeros((16,), jnp.float32)

    def zfill(r, carry):
        for f in range(D // 16):
            zbuf[r, pl.ds(16 * f, 16)] = zero16
        return carry

    lax.fori_loop(0, 40, zfill, 0)

    def zfillv(i, carry):
        zbufv[pl.ds(i * 16, 16)] = zero16
        return carry

    lax.fori_loop(0, ZR // 16, zfillv, 0)
    zbufv[pl.ds(ZR - 16, 16)] = zero16

    @pl.when(s < 10)
    def _():
        r0 = s * ZR
        for i in range(ZR // 40):
            pltpu.sync_copy(zbuf, u_sh.at[pl.ds(r0 + 40 * i, 40)])
        pltpu.sync_copy(zbufv, dn_sh.at[pl.ds(r0, ZR)])
        pltpu.sync_copy(zbufv, ss_sh.at[pl.ds(r0, ZR)])
        pltpu.sync_copy(zbufv, cnt_sh.at[pl.ds(r0, ZR)])

    for i in range(CH // 16):
        ones_v[pl.ds(16 * i, 16)] = jnp.full((16,), 1.0, jnp.float32)

    plsc.subcore_barrier()

    ebase = s * EPT
    exbase = c * E + ebase

    def start_g(x, ag, bg, rows, sg):
        def go(tas, tad, th):
            pltpu.async_copy(tas.at[srcb.at[x]], ag, sg)
            pltpu.async_copy(tad.at[dstb.at[x]], bg, sg)
            pltpu.async_copy(th.at[srcb.at[x]], rows, sg)

        @pl.when(c == 0)
        def _():
            go(as0, ad0, h0)

        @pl.when(c == 1)
        def _():
            go(as1, ad1, h1)

    def wait_g(x, ag, bg, rows, sg):
        def wg(tas, tad, th):
            pltpu.make_async_copy(tas.at[srcb.at[x]], ag, sg).wait()
            pltpu.make_async_copy(tad.at[dstb.at[x]], bg, sg).wait()
            pltpu.make_async_copy(th.at[srcb.at[x]], rows, sg).wait()

        @pl.when(c == 0)
        def _():
            wg(as0, ad0, h0)

        @pl.when(c == 1)
        def _():
            wg(as1, ad1, h1)

    def compute_ex(ag, bg, att):
        for i in range(CH // 16):
            sl = pl.ds(16 * i, 16)
            a = ag[sl] + bg[sl]
            a = jnp.maximum(a, 0.2 * a)
            att[sl] = jnp.exp(a)

    def mul_rows(rows, att):
        def mbody(g, carry2):
            ev16 = att[pl.ds(16 * g, 16)]
            for lane in range(16):
                e = ev16[lane]
                r = 16 * g + lane
                for f in range(D // 16):
                    sl = pl.ds(16 * f, 16)
                    rows[r, sl] = rows[r, sl] * e
            return carry2

        lax.fori_loop(0, CH // 16, mbody, 0)

    def scatter_sc(bgl, x, rows, att):
        pltpu.sync_copy(rows, u_sh.at[dstb.at[x]], add=True)
        pltpu.sync_copy(att, dn_sh.at[dstb.at[x]], add=True)
        pltpu.sync_copy(att, exh.at[pl.ds(exbase + bgl * CH, CH)])

        @pl.when(c == 0)
        def _():
            pltpu.sync_copy(ones_v, cnt_sh.at[srcb.at[x]], add=True)

    def block(b, carry):
        pltpu.sync_copy(src_h.at[s * NBLK + b], srcb)
        pltpu.sync_copy(dst_h.at[s * NBLK + b], dstb)
        base = b * BLK
        start_g(0, aga, bga, rows_a, sg0)

        def pair(k, carry2):
            a = 2 * k
            bl = 2 * k + 1
            wait_g(a, aga, bga, rows_a, sg0)
            compute_ex(aga, bga, atta)
            start_g(bl, agb, bgb, rows_b, sg1)
            mul_rows(rows_a, atta)
            scatter_sc(base + a, a, rows_a, atta)
            wait_g(bl, agb, bgb, rows_b, sg1)
            compute_ex(agb, bgb, attb)

            @pl.when(k < BLK // 2 - 1)
            def _():
                start_g(a + 2, aga, bga, rows_a, sg0)

            mul_rows(rows_b, attb)
            scatter_sc(base + bl, bl, rows_b, attb)
            return carry2

        lax.fori_loop(0, BLK // 2, pair, 0)
        return carry

    lax.fori_loop(0, NBLK, block, 0)
    plsc.subcore_barrier()

    def ex_start(x, buf, sg):
        pltpu.async_copy(exh.at[pl.ds(exbase + x * CH, CH)], buf, sg)

    def ex_wait(x, buf, sg):
        pltpu.make_async_copy(
            exh.at[pl.ds(exbase + x * CH, CH)], buf, sg).wait()

    def block2(b, carry):
        pltpu.sync_copy(src_h.at[s * NBLK + b], srcb)
        pltpu.sync_copy(dst_h.at[s * NBLK + b], dstb)
        base = b * BLK
        ex_start(base, exb, sg0)

        def p2chunk(x, buf, sg):
            ex_wait(base + x, buf, sg)
            pltpu.async_copy(dn_sh.at[dstb.at[x]], bga, sem).wait()
            for i in range(CH // 16):
                sl = pl.ds(16 * i, 16)
                atta[sl] = buf[sl] / (bga[sl] + 1e-16)
            pltpu.sync_copy(atta, ss_sh.at[srcb.at[x]], add=True)

        def pair2(k, carry2):
            a = 2 * k
            bl = 2 * k + 1
            ex_start(base + bl, agb, sg1)
            p2chunk(a, exb, sg0)

            @pl.when(k < BLK // 2 - 1)
            def _():
                ex_start(base + a + 2, exb, sg0)

            p2chunk(bl, agb, sg1)
            return carry2

        lax.fori_loop(0, BLK // 2, pair2, 0)
        return carry

    lax.fori_loop(0, NBLK, block2, 0)
    plsc.subcore_barrier()

    @pl.when(s < 10)
    def _():
        r0 = s * ZR

        def d_out(o):
            def dchunk(i, carry):
                sl = pl.ds(r0 + 40 * i, 40)
                pltpu.sync_copy(u_sh.at[sl], zbuf)
                pltpu.sync_copy(zbuf, o.at[sl])
                return carry

            lax.fori_loop(0, ZR // 40, dchunk, 0)

        def d_vec(sh, o):
            sl = pl.ds(r0, ZR)
            pltpu.sync_copy(sh.at[sl], zbufv)
            pltpu.sync_copy(zbufv, o.at[sl])

        pick(d_out)(out0, out1)
        pick(lambda o: d_vec(dn_sh, o))(dn0, dn1)
        pick(lambda o: d_vec(ss_sh, o))(ss0, ss1)

        @pl.when(c == 0)
        def _():
            d_vec(cnt_sh, cnt)


def _sc_call(h0, h1, as0, ad0, as1, ad1, src_r, dst_r):
    mesh = plsc.VectorSubcoreMesh(core_axis_name="c", subcore_axis_name="s",
                                  num_cores=NC, num_subcores=NS)
    f32 = jnp.float32
    out_type = [
        jax.ShapeDtypeStruct((N, D), f32),
        jax.ShapeDtypeStruct((N, D), f32),
        jax.ShapeDtypeStruct((N,), f32),
        jax.ShapeDtypeStruct((N,), f32),
        jax.ShapeDtypeStruct((N,), f32),
        jax.ShapeDtypeStruct((N,), f32),
        jax.ShapeDtypeStruct((N,), f32),
        jax.ShapeDtypeStruct((2 * E,), f32),
    ]
    scratch = [
        pltpu.VMEM_SHARED((N, D), f32),
        pltpu.VMEM_SHARED((N,), f32),
        pltpu.VMEM_SHARED((N,), f32),
        pltpu.VMEM_SHARED((N,), f32),
        pltpu.VMEM((BLK, CH), jnp.int32),
        pltpu.VMEM((BLK, CH), jnp.int32),
        pltpu.VMEM((CH, D), f32),
        pltpu.VMEM((CH, D), f32),
        pltpu.VMEM((CH,), f32),
        pltpu.VMEM((CH,), f32),
        pltpu.VMEM((CH,), f32),
        pltpu.VMEM((CH,), f32),
        pltpu.VMEM((CH,), f32),
        pltpu.VMEM((CH,), f32),
        pltpu.VMEM((CH,), f32),
        pltpu.VMEM((CH,), f32),
        pltpu.VMEM((40, D), f32),
        pltpu.VMEM((ZR,), f32),
        pltpu.SemaphoreType.DMA,
        pltpu.SemaphoreType.DMA,
        pltpu.SemaphoreType.DMA,
    ]
    fn = pl.kernel(_sc_body, out_type=out_type, mesh=mesh,
                   scratch_types=scratch)
    return fn(h0, h1, as0, ad0, as1, ad1, src_r, dst_r)


def _tc2_body(o0_ref, o1_ref, dn0_ref, dn1_ref, ss0_ref, ss1_ref, cnt_ref,
              b0_ref, b1_ref, feat_ref, scores_ref):
    eps = 1e-16
    o0 = o0_ref[...] / (dn0_ref[...] + eps) + b0_ref[...]
    o1 = o1_ref[...] / (dn1_ref[...] + eps) + b1_ref[...]
    f0 = jnp.maximum(o0, 0.01 * o0)
    f1 = jnp.maximum(o1, 0.01 * o1)
    feat_ref[...] = f0 + f1
    ssum = ss0_ref[...] + ss1_ref[...]
    scores_ref[...] = ssum / jnp.maximum(cnt_ref[...], 1.0)


def _tc2(out0, out1, dn0, dn1, ss0, ss1, cnt, b0, b1):
    f32 = jnp.float32
    return pl.pallas_call(
        _tc2_body,
        out_shape=[
            jax.ShapeDtypeStruct((N, D), f32),
            jax.ShapeDtypeStruct((N, 1), f32),
        ],
    )(out0, out1, dn0, dn1, ss0, ss1, cnt, b0, b1)


def kernel(x, edge_index, W0, a_src0, a_dst0, b0, W1, a_src1, a_dst1, b1):
    src_r = edge_index[0].reshape(NS * NBLK, BLK, CH)
    dst_r = edge_index[1].reshape(NS * NBLK, BLK, CH)
    h0, h1, scal = _tc1(x, W0, W1,
                        a_src0.reshape(1, D), a_dst0.reshape(1, D),
                        a_src1.reshape(1, D), a_dst1.reshape(1, D))
    as0, ad0, as1, ad1 = scal[0], scal[1], scal[2], scal[3]
    out0, out1, dn0, dn1, ss0, ss1, cnt, _ = _sc_call(
        h0, h1, as0, ad0, as1, ad1, src_r, dst_r)
    feat, scores = _tc2(out0, out1,
                        dn0.reshape(N, 1), dn1.reshape(N, 1),
                        ss0.reshape(N, 1), ss1.reshape(N, 1),
                        cnt.reshape(N, 1),
                        b0.reshape(1, D), b1.reshape(1, D))
    return (feat, scores.reshape(N))
